# Initial kernel scaffold; baseline (speedup 1.0000x reference)
import jax
import jax.numpy as jnp
from jax.experimental import pallas as pl


def kernel(scores):
    raise NotImplementedError("write your pallas kernel here")


# speedup vs baseline: 4.4148x; 4.4148x over previous
import functools

import jax
import jax.numpy as jnp
from jax.experimental import pallas as pl
from jax.experimental.pallas import tpu as pltpu

NUM_ITERS = 10
STRIP = 512


def _iter_kernel(nstrips, s_ref, r_ref, c_ref, w_ref, acc_ref):
    t = pl.program_id(0)
    i = pl.program_id(1)

    @pl.when(jnp.logical_and(t == 0, i == 0))
    def _():
        w_ref[...] = jnp.ones_like(w_ref)

    @pl.when(i == 0)
    def _():
        acc_ref[...] = jnp.zeros_like(acc_ref)

    e = jnp.exp(s_ref[...])
    r = 1.0 / jnp.sum(e * w_ref[...], axis=1, keepdims=True)
    r_ref[...] = r
    acc_ref[...] += jnp.sum(e * r, axis=0, keepdims=True)

    @pl.when(i == nstrips - 1)
    def _():
        w = 1.0 / acc_ref[...]
        w_ref[...] = w
        c_ref[...] = w


def _finalize_kernel(s_ref, r_ref, c_ref, o_ref):
    o_ref[...] = jnp.exp(s_ref[...]) * r_ref[...] * c_ref[...]


def kernel(scores: jnp.ndarray) -> jnp.ndarray:
    m, n = scores.shape
    strip = min(STRIP, m)
    nstrips = m // strip

    r, c = pl.pallas_call(
        functools.partial(_iter_kernel, nstrips),
        grid=(NUM_ITERS, nstrips),
        in_specs=[pl.BlockSpec((strip, n), lambda t, i: (i, 0))],
        out_specs=[
            pl.BlockSpec((strip, 1), lambda t, i: (i, 0)),
            pl.BlockSpec((1, n), lambda t, i: (0, 0)),
        ],
        out_shape=[
            jax.ShapeDtypeStruct((m, 1), jnp.float32),
            jax.ShapeDtypeStruct((1, n), jnp.float32),
        ],
        scratch_shapes=[
            pltpu.VMEM((1, n), jnp.float32),
            pltpu.VMEM((1, n), jnp.float32),
        ],
        compiler_params=pltpu.CompilerParams(
            dimension_semantics=("arbitrary", "arbitrary"),
            vmem_limit_bytes=50 * 1024 * 1024,
        ),
        name="sinkhorn_iters",
    )(scores)

    fstrip = min(256, m)
    fnstrips = m // fstrip
    out = pl.pallas_call(
        _finalize_kernel,
        grid=(fnstrips,),
        in_specs=[
            pl.BlockSpec((fstrip, n), lambda i: (i, 0)),
            pl.BlockSpec((fstrip, 1), lambda i: (i, 0)),
            pl.BlockSpec((1, n), lambda i: (0, 0)),
        ],
        out_specs=pl.BlockSpec((fstrip, n), lambda i: (i, 0)),
        out_shape=jax.ShapeDtypeStruct((m, n), jnp.float32),
        compiler_params=pltpu.CompilerParams(
            dimension_semantics=("parallel",),
            vmem_limit_bytes=50 * 1024 * 1024,
        ),
        name="sinkhorn_finalize",
    )(scores, r, c)
    return out
